# baseline (speedup 1.0000x reference)
import functools

import jax
import jax.numpy as jnp
from jax import lax
from jax.experimental import pallas as pl
from jax.experimental.pallas import tpu as pltpu
from jax.experimental.pallas import tpu_sc as plsc

_B = 4
_S = 4096
_D = 2048
_NW = 32
_SROWS = _S // _NW
_C = 8
_CHUNK = _C * _D
_NCH = _SROWS // _C
_T = _NCH * _B
_LANES = 16


def _sc_body(x_hbm, pos_hbm, out_hbm,
             xb0, xb1, pb0, pb1, ob0, ob1,
             sx0, sx1, sp0, sp1, so0, so1):
    wid = lax.axis_index("s") * 2 + lax.axis_index("c")
    s0 = wid * _SROWS
    xbs, pbs, obs = (xb0, xb1), (pb0, pb1), (ob0, ob1)
    sxs, sps, sos = (sx0, sx1), (sp0, sp1), (so0, so1)

    def x_off(c, b):
        return (b * _S + s0 + c * _C) * _D

    pltpu.async_copy(pos_hbm.at[pl.ds(s0 * _D, _CHUNK)], pb0, sp0)
    pltpu.async_copy(x_hbm.at[pl.ds(x_off(0, 0), _CHUNK)], xb0, sx0)

    for c in range(_NCH):
        for b in range(_B):
            t = c * _B + b
            k = t % 2
            if t + 1 < _T:
                c1, b1 = divmod(t + 1, _B)
                pltpu.async_copy(
                    x_hbm.at[pl.ds(x_off(c1, b1), _CHUNK)],
                    xbs[(t + 1) % 2], sxs[(t + 1) % 2])
            if b == 0 and c + 1 < _NCH:
                pltpu.async_copy(
                    pos_hbm.at[pl.ds((s0 + (c + 1) * _C) * _D, _CHUNK)],
                    pbs[(c + 1) % 2], sps[(c + 1) % 2])
            pltpu.make_async_copy(
                x_hbm.at[pl.ds(0, _CHUNK)], xbs[k], sxs[k]).wait()
            if b == 0:
                pltpu.make_async_copy(
                    pos_hbm.at[pl.ds(0, _CHUNK)], pbs[c % 2], sps[c % 2]).wait()
            if t >= 2:
                pltpu.make_async_copy(
                    obs[k], out_hbm.at[pl.ds(0, _CHUNK)], sos[k]).wait()

            xbuf, pbuf, obuf = xbs[k], pbs[c % 2], obs[k]

            @plsc.parallel_loop(0, _CHUNK, _LANES, unroll=16)
            def _vec_body(v, xbuf=xbuf, pbuf=pbuf, obuf=obuf):
                obuf[pl.ds(v, _LANES)] = (
                    xbuf[pl.ds(v, _LANES)] + pbuf[pl.ds(v, _LANES)])
            pltpu.async_copy(obuf, out_hbm.at[pl.ds(x_off(c, b), _CHUNK)], sos[k])

    for k in ((_T - 2) % 2, (_T - 1) % 2):
        pltpu.make_async_copy(
            obs[k], out_hbm.at[pl.ds(0, _CHUNK)], sos[k]).wait()


def kernel(input, pos_table):
    x_flat = input.reshape(-1)
    pos_flat = pos_table.reshape(-1)
    mesh = plsc.VectorSubcoreMesh(core_axis_name="c", subcore_axis_name="s")
    sc_add = functools.partial(
        pl.kernel,
        mesh=mesh,
        out_type=jax.ShapeDtypeStruct((_B * _S * _D,), jnp.float32),
        scratch_types=[
            pltpu.VMEM((_CHUNK,), jnp.float32),
            pltpu.VMEM((_CHUNK,), jnp.float32),
            pltpu.VMEM((_CHUNK,), jnp.float32),
            pltpu.VMEM((_CHUNK,), jnp.float32),
            pltpu.VMEM((_CHUNK,), jnp.float32),
            pltpu.VMEM((_CHUNK,), jnp.float32),
            pltpu.SemaphoreType.DMA,
            pltpu.SemaphoreType.DMA,
            pltpu.SemaphoreType.DMA,
            pltpu.SemaphoreType.DMA,
            pltpu.SemaphoreType.DMA,
            pltpu.SemaphoreType.DMA,
        ],
    )(_sc_body)
    out = sc_add(x_flat, pos_flat)
    return out.reshape(input.shape)


# speedup vs baseline: 4.3161x; 4.3161x over previous
import jax
import jax.numpy as jnp
from jax.experimental import pallas as pl
from jax.experimental.pallas import tpu as pltpu

_BS = 1024


def _add_kernel(x_ref, pos_ref, o_ref):
    o_ref[...] = x_ref[...] + pos_ref[...]


def kernel(input, pos_table):
    batch, seq_len, d_model = input.shape
    grid = (seq_len // _BS, batch)
    return pl.pallas_call(
        _add_kernel,
        grid=grid,
        in_specs=[
            pl.BlockSpec((1, _BS, d_model), lambda s, b: (b, s, 0)),
            pl.BlockSpec((_BS, d_model), lambda s, b: (s, 0)),
        ],
        out_specs=pl.BlockSpec((1, _BS, d_model), lambda s, b: (b, s, 0)),
        out_shape=jax.ShapeDtypeStruct(input.shape, input.dtype),
        compiler_params=pltpu.CompilerParams(
            dimension_semantics=("parallel", "parallel"),
        ),
    )(input, pos_table)
